# baseline (speedup 1.0000x reference)
import functools

import jax
import jax.numpy as jnp
from jax import lax
from jax.experimental import pallas as pl
from jax.experimental.pallas import tpu as pltpu
from jax.experimental.pallas import tpu_sc as plsc

NC = 2
NS = 16
NW = NC * NS
LANES = 16


def _make_kernel(N, V, H, S, B):
    CB = 512
    QB = B // CB
    CT = CB // 128
    HT = H // 8
    n_units = S * QB
    per_w = n_units // NW
    assert n_units % NW == 0 and per_w % 2 == 0
    NBUF = 2
    idx_per_w = per_w * CB
    l_stride = HT * (B // 128) * 1024
    r_stride = (B // 128) * 1024

    mesh = plsc.VectorSubcoreMesh(core_axis_name="c", subcore_axis_name="s")

    @functools.partial(
        pl.kernel,
        out_type=jax.ShapeDtypeStruct((N * H,), jnp.float32),
        mesh=mesh,
        compiler_params=pltpu.CompilerParams(use_tc_tiling_on_sc=False,
                                             needs_layout_passes=False),
        scratch_types=dict(
            idx_v=pltpu.VMEM((idx_per_w,), jnp.int32),
            pos_v=pltpu.VMEM((S * H,), jnp.float32),
            rows_v=pltpu.VMEM((NBUF, CB, H), jnp.float32),
            obuf=pltpu.VMEM((NBUF, CB * H), jnp.float32),
            gsems=[pltpu.SemaphoreType.DMA] * NBUF,
            osems=[pltpu.SemaphoreType.DMA] * NBUF,
        ),
    )
    def body(idx_hbm, emb_hbm, pos_hbm, out_hbm, idx_v, pos_v, rows_v, obuf,
             gsems, osems):
        wid = lax.axis_index("s") * NC + lax.axis_index("c")
        u0 = wid * per_w

        pltpu.sync_copy(idx_hbm.at[pl.ds(u0 * CB, idx_per_w)], idx_v)
        pltpu.sync_copy(pos_hbm, pos_v)

        @plsc.parallel_loop(0, idx_per_w, step=LANES, unroll=4)
        def permute_idx(j):
            v = idx_v[pl.ds(j, LANES)]
            a = v & 511
            idx_v[pl.ds(j, LANES)] = (v - a) + ((v & 127) << 2) + (a >> 7)

        iota = lax.iota(jnp.int32, LANES)
        rot = [(iota + s) & 15 for s in range(LANES)]
        h_vecs, addr_half = [], []
        for half in range(2):
            h = iota + 16 * half
            h_vecs.append(h)
            addr_half.append((h >> 3) * (1024 * CT) + (h & 7) * 128)

        def out_pieces(u, bb, sem):
            l, q = u // QB, u % QB
            base = l * l_stride + q * (CT * 1024)
            for rt in range(HT):
                yield (obuf.at[bb, pl.ds(rt * (CT * 1024), CT * 1024)],
                       out_hbm.at[pl.ds(base + rt * r_stride, CT * 1024)],
                       sem)

        def start_gather(u, bb):
            pltpu.async_copy(
                emb_hbm.at[idx_v.at[pl.ds((u - u0) * CB, CB)]],
                rows_v.at[bb], gsems[bb])

        for bb in range(NBUF):
            start_gather(u0 + bb, bb)

        @pl.loop(0, per_w, step=NBUF)
        def group(k0):
            for bb in range(NBUF):
                k = k0 + bb
                u = u0 + k
                l = u // QB

                pltpu.make_async_copy(
                    emb_hbm.at[idx_v.at[pl.ds(k * CB, CB)]],
                    rows_v.at[bb], gsems[bb]).wait()

                @pl.when(k >= NBUF)
                def _():
                    for s, d, m in out_pieces(u - NBUF, bb, osems[bb]):
                        pltpu.make_async_copy(s, d, m).wait()

                pos2 = [pos_v[pl.ds(l * H, LANES)],
                        pos_v[pl.ds(l * H + LANES, LANES)]]
                ob = obuf.at[bb]
                rows2d = rows_v.at[bb]

                for half in range(2):
                    p, hv, ah = pos2[half], h_vecs[half], addr_half[half]

                    @plsc.parallel_loop(0, CB, step=LANES, unroll=2)
                    def transform(b0):
                        goff = (b0 >> 7) * 896
                        for s in range(LANES):
                            bv = rot[s] + b0
                            val = plsc.load_gather(rows2d, [bv, hv]) + p
                            plsc.store_scatter(ob, [ah + bv + goff], val)

                for s, d, m in out_pieces(u, bb, osems[bb]):
                    pltpu.async_copy(s, d, m)

                @pl.when(k + NBUF < per_w)
                def _():
                    start_gather(u + NBUF, bb)

        for bb in range(NBUF):
            u_last = u0 + per_w - NBUF + bb
            for s, d, m in out_pieces(u_last, bb, osems[bb]):
                pltpu.make_async_copy(s, d, m).wait()

    return body


def _tc_transpose_body(in_ref, out_ref):
    for c in range(4):
        out_ref[:, 32 * c:32 * (c + 1)] = in_ref[:, 128 * c:128 * (c + 1)].T


def _make_tc_transpose(V, H):
    WB = 512
    grid = (V + WB - 1) // WB
    return pl.pallas_call(
        _tc_transpose_body,
        grid=(grid,),
        in_specs=[pl.BlockSpec((H, WB), lambda k: (0, k))],
        out_specs=pl.BlockSpec((WB * H // 128, 128), lambda k: (k, 0)),
        out_shape=jax.ShapeDtypeStruct((grid * WB * H // 128, 128),
                                       jnp.float32),
    )


def kernel(x, emb, pos_emb):
    B, S = x.shape
    V, H = emb.shape
    N = B * S
    idx = x.T.reshape(N)
    emb128 = _make_tc_transpose(V, H)(emb.T)
    emb_lin = emb128.reshape(emb128.shape[0] * 128 // H, H)
    posf = pos_emb.reshape(S * H)
    fn = _make_kernel(N, V, H, S, B)
    outf = fn(idx, emb_lin, posf)
    out5 = outf.reshape(S, H // 8, B // 128, 8, 128)
    out = out5.transpose(0, 1, 3, 2, 4).reshape(S, H, B).transpose(2, 0, 1)
    return out


# speedup vs baseline: 3.2134x; 3.2134x over previous
import functools

import jax
import jax.numpy as jnp
from jax import lax
from jax.experimental import pallas as pl
from jax.experimental.pallas import tpu as pltpu
from jax.experimental.pallas import tpu_sc as plsc

NC = 2
NS = 16
NW = NC * NS
LANES = 16


def _make_kernel(N, V, H, S, B):
    CB = 512
    QB = B // CB
    CT = CB // 128
    HT = H // 8
    n_units = S * QB
    per_w = n_units // NW
    assert n_units % NW == 0 and per_w % 2 == 0
    NBUF = 2
    idx_per_w = per_w * CB
    l_stride = HT * (B // 128) * 1024
    r_stride = (B // 128) * 1024

    mesh = plsc.VectorSubcoreMesh(core_axis_name="c", subcore_axis_name="s")

    @functools.partial(
        pl.kernel,
        out_type=jax.ShapeDtypeStruct((N * H,), jnp.float32),
        mesh=mesh,
        compiler_params=pltpu.CompilerParams(use_tc_tiling_on_sc=False,
                                             needs_layout_passes=False),
        scratch_types=dict(
            idx_v=pltpu.VMEM((idx_per_w,), jnp.int32),
            pos_v=pltpu.VMEM((S * H,), jnp.float32),
            rows_v=pltpu.VMEM((NBUF, CB, H), jnp.float32),
            obuf=pltpu.VMEM((NBUF, CB * H), jnp.float32),
            gsems=[pltpu.SemaphoreType.DMA] * NBUF,
            osems=[pltpu.SemaphoreType.DMA] * NBUF,
        ),
    )
    def body(idx_hbm, emb_hbm, pos_hbm, out_hbm, idx_v, pos_v, rows_v, obuf,
             gsems, osems):
        wid = lax.axis_index("s") * NC + lax.axis_index("c")
        u0 = wid * per_w

        pltpu.sync_copy(idx_hbm.at[pl.ds(u0 * CB, idx_per_w)], idx_v)
        pltpu.sync_copy(pos_hbm, pos_v)

        @plsc.parallel_loop(0, idx_per_w, step=LANES, unroll=4)
        def permute_idx(j):
            v = idx_v[pl.ds(j, LANES)]
            a = v & 511
            idx_v[pl.ds(j, LANES)] = (v - a) + ((v & 127) << 2) + (a >> 7)

        iota = lax.iota(jnp.int32, LANES)
        rot = [(iota + s) & 15 for s in range(LANES)]
        h_vecs, addr_half = [], []
        for half in range(2):
            h = iota + 16 * half
            h_vecs.append(h)
            addr_half.append((h >> 3) * (1024 * CT) + (h & 7) * 128)

        def out_pieces(u, bb, sem):
            l, q = u // QB, u % QB
            base = l * l_stride + q * (CT * 1024)
            for rt in range(HT):
                yield (obuf.at[bb, pl.ds(rt * (CT * 1024), CT * 1024)],
                       out_hbm.at[pl.ds(base + rt * r_stride, CT * 1024)],
                       sem)

        def start_gather(u, bb):
            pltpu.async_copy(
                emb_hbm.at[idx_v.at[pl.ds((u - u0) * CB, CB)]],
                rows_v.at[bb], gsems[bb])

        for bb in range(NBUF):
            start_gather(u0 + bb, bb)

        @pl.loop(0, per_w, step=NBUF)
        def group(k0):
            for bb in range(NBUF):
                k = k0 + bb
                u = u0 + k
                l = u // QB

                pltpu.make_async_copy(
                    emb_hbm.at[idx_v.at[pl.ds(k * CB, CB)]],
                    rows_v.at[bb], gsems[bb]).wait()

                @pl.when(k >= NBUF)
                def _():
                    for s, d, m in out_pieces(u - NBUF, bb, osems[bb]):
                        pltpu.make_async_copy(s, d, m).wait()

                pos2 = [pos_v[pl.ds(l * H, LANES)],
                        pos_v[pl.ds(l * H + LANES, LANES)]]
                ob = obuf.at[bb]
                rows2d = rows_v.at[bb]

                for half in range(2):
                    p, hv, ah = pos2[half], h_vecs[half], addr_half[half]

                    @plsc.parallel_loop(0, CB, step=LANES, unroll=2)
                    def transform(b0):
                        goff = (b0 >> 7) * 896
                        for s in range(LANES):
                            bv = rot[s] + b0
                            val = plsc.load_gather(rows2d, [bv, hv]) + p
                            plsc.store_scatter(ob, [ah + bv + goff], val)

                for s, d, m in out_pieces(u, bb, osems[bb]):
                    pltpu.async_copy(s, d, m)

                @pl.when(k + NBUF < per_w)
                def _():
                    start_gather(u + NBUF, bb)

        for bb in range(NBUF):
            u_last = u0 + per_w - NBUF + bb
            for s, d, m in out_pieces(u_last, bb, osems[bb]):
                pltpu.make_async_copy(s, d, m).wait()

    return body


_WB = 8192


def _tc_transpose_body(in_ref, out_ref):
    for c in range(_WB // 128):
        out_ref[128 * (c // 4):128 * (c // 4) + 128,
                32 * (c % 4):32 * (c % 4) + 32] = \
            in_ref[:, 128 * c:128 * (c + 1)].T


def _make_tc_transpose(V, H):
    grid = (V + _WB - 1) // _WB
    return pl.pallas_call(
        _tc_transpose_body,
        grid=(grid,),
        in_specs=[pl.BlockSpec((H, _WB), lambda k: (0, k))],
        out_specs=pl.BlockSpec((_WB * H // 128, 128), lambda k: (k, 0)),
        out_shape=jax.ShapeDtypeStruct((grid * _WB * H // 128, 128),
                                       jnp.float32),
    )


def kernel(x, emb, pos_emb):
    B, S = x.shape
    V, H = emb.shape
    N = B * S
    idx = x.T.reshape(N)
    emb128 = _make_tc_transpose(V, H)(emb.T)
    emb_lin = emb128.reshape(emb128.shape[0] * 128 // H, H)
    posf = pos_emb.reshape(S * H)
    fn = _make_kernel(N, V, H, S, B)
    outf = fn(idx, emb_lin, posf)
    out5 = outf.reshape(S, H // 8, B // 128, 8, 128)
    out = out5.transpose(0, 1, 3, 2, 4).reshape(S, H, B).transpose(2, 0, 1)
    return out


# speedup vs baseline: 3.2155x; 1.0007x over previous
import functools

import jax
import jax.numpy as jnp
from jax import lax
from jax.experimental import pallas as pl
from jax.experimental.pallas import tpu as pltpu
from jax.experimental.pallas import tpu_sc as plsc

NC = 2
NS = 16
NW = NC * NS
LANES = 16


def _make_kernel(N, V, H, S, B):
    CB = 512
    QB = B // CB
    CT = CB // 128
    HT = H // 8
    n_units = S * QB
    per_w = n_units // NW
    assert n_units % NW == 0 and per_w % 2 == 0
    NBUF = 2
    idx_per_w = per_w * CB
    l_stride = HT * (B // 128) * 1024
    r_stride = (B // 128) * 1024

    mesh = plsc.VectorSubcoreMesh(core_axis_name="c", subcore_axis_name="s")

    @functools.partial(
        pl.kernel,
        out_type=jax.ShapeDtypeStruct((N * H,), jnp.float32),
        mesh=mesh,
        compiler_params=pltpu.CompilerParams(use_tc_tiling_on_sc=False,
                                             needs_layout_passes=False),
        scratch_types=dict(
            idx_v=pltpu.VMEM((idx_per_w,), jnp.int32),
            pos_v=pltpu.VMEM((S * H,), jnp.float32),
            rows_v=pltpu.VMEM((NBUF, CB, H), jnp.float32),
            obuf=pltpu.VMEM((NBUF, CB * H), jnp.float32),
            gsems=[pltpu.SemaphoreType.DMA] * NBUF,
            osems=[pltpu.SemaphoreType.DMA] * NBUF,
        ),
    )
    def body(idx_hbm, emb_hbm, pos_hbm, out_hbm, idx_v, pos_v, rows_v, obuf,
             gsems, osems):
        wid = lax.axis_index("s") * NC + lax.axis_index("c")
        u0 = wid * per_w

        pltpu.sync_copy(idx_hbm.at[pl.ds(u0 * CB, idx_per_w)], idx_v)
        pltpu.sync_copy(pos_hbm, pos_v)

        @plsc.parallel_loop(0, idx_per_w, step=LANES, unroll=4)
        def permute_idx(j):
            v = idx_v[pl.ds(j, LANES)]
            a = v & 511
            idx_v[pl.ds(j, LANES)] = (v - a) + ((v & 127) << 2) + (a >> 7)

        iota = lax.iota(jnp.int32, LANES)
        rot = [(iota + s) & 15 for s in range(LANES)]
        h_vecs, addr_half = [], []
        for half in range(2):
            h = iota + 16 * half
            h_vecs.append(h)
            addr_half.append((h >> 3) * (1024 * CT) + (h & 7) * 128)

        def out_pieces(u, bb, sem):
            l, q = u // QB, u % QB
            base = l * l_stride + q * (CT * 1024)
            for rt in range(HT):
                yield (obuf.at[bb, pl.ds(rt * (CT * 1024), CT * 1024)],
                       out_hbm.at[pl.ds(base + rt * r_stride, CT * 1024)],
                       sem)

        def start_gather(u, bb):
            pltpu.async_copy(
                emb_hbm.at[idx_v.at[pl.ds((u - u0) * CB, CB)]],
                rows_v.at[bb], gsems[bb])

        for bb in range(NBUF):
            start_gather(u0 + bb, bb)

        @pl.loop(0, per_w, step=NBUF)
        def group(k0):
            for bb in range(NBUF):
                k = k0 + bb
                u = u0 + k
                l = u // QB

                pltpu.make_async_copy(
                    emb_hbm.at[idx_v.at[pl.ds(k * CB, CB)]],
                    rows_v.at[bb], gsems[bb]).wait()

                @pl.when(k >= NBUF)
                def _():
                    for s, d, m in out_pieces(u - NBUF, bb, osems[bb]):
                        pltpu.make_async_copy(s, d, m).wait()

                pos2 = [pos_v[pl.ds(l * H, LANES)],
                        pos_v[pl.ds(l * H + LANES, LANES)]]
                ob = obuf.at[bb]
                rows2d = rows_v.at[bb]

                for half in range(2):
                    p, hv, ah = pos2[half], h_vecs[half], addr_half[half]

                    @plsc.parallel_loop(0, CB, step=LANES, unroll=2)
                    def transform(b0):
                        goff = (b0 >> 7) * 896
                        for s in range(LANES):
                            bv = rot[s] + b0
                            val = plsc.load_gather(rows2d, [bv, hv]) + p
                            plsc.store_scatter(ob, [ah + bv + goff], val)

                for s, d, m in out_pieces(u, bb, osems[bb]):
                    pltpu.async_copy(s, d, m)

                @pl.when(k + NBUF < per_w)
                def _():
                    start_gather(u + NBUF, bb)

        for bb in range(NBUF):
            u_last = u0 + per_w - NBUF + bb
            for s, d, m in out_pieces(u_last, bb, osems[bb]):
                pltpu.make_async_copy(s, d, m).wait()

    return body


_WB = 8192


def _tc_transpose_body(in_ref, out_ref):
    for g in range(_WB // 512):
        parts = [in_ref[:, 128 * (4 * g + i):128 * (4 * g + i) + 128].T
                 for i in range(4)]
        out_ref[128 * g:128 * g + 128, :] = jnp.concatenate(parts, axis=1)


def _make_tc_transpose(V, H):
    grid = (V + _WB - 1) // _WB
    return pl.pallas_call(
        _tc_transpose_body,
        grid=(grid,),
        in_specs=[pl.BlockSpec((H, _WB), lambda k: (0, k))],
        out_specs=pl.BlockSpec((_WB * H // 128, 128), lambda k: (k, 0)),
        out_shape=jax.ShapeDtypeStruct((grid * _WB * H // 128, 128),
                                       jnp.float32),
    )


def kernel(x, emb, pos_emb):
    B, S = x.shape
    V, H = emb.shape
    N = B * S
    idx = x.T.reshape(N)
    emb128 = _make_tc_transpose(V, H)(emb.T)
    emb_lin = emb128.reshape(emb128.shape[0] * 128 // H, H)
    posf = pos_emb.reshape(S * H)
    fn = _make_kernel(N, V, H, S, B)
    outf = fn(idx, emb_lin, posf)
    out5 = outf.reshape(S, H // 8, B // 128, 8, 128)
    out = out5.transpose(0, 1, 3, 2, 4).reshape(S, H, B).transpose(2, 0, 1)
    return out
